# baseline (speedup 1.0000x reference)
import jax
import jax.numpy as jnp
from jax.experimental import pallas as pl
from jax.experimental.pallas import tpu as pltpu

_MIB = 1024 * 1024


def _matmul_kernel(x_ref, w_ref, o_ref, w_bf_ref):
    x_bf = x_ref[...].astype(jnp.bfloat16)

    @pl.when(pl.program_id(0) == 0)
    def _():
        w_bf_ref[...] = w_ref[...].T.astype(jnp.bfloat16)
        o_ref[...] = jax.lax.dot_general(
            x_bf,
            w_ref[...].astype(jnp.bfloat16),
            dimension_numbers=(((1,), (1,)), ((), ())),
            preferred_element_type=jnp.float32,
        )

    @pl.when(pl.program_id(0) > 0)
    def _():
        o_ref[...] = jnp.dot(
            x_bf,
            w_bf_ref[...],
            preferred_element_type=jnp.float32,
        )


def kernel(x, weight):
    M, K = x.shape
    N = weight.shape[0]
    out_dtype = x.dtype

    tm = 512
    grid_m = M // tm

    footprint = K * N * 4 + K * N * 2 + 2 * tm * (K + N) * 4

    return pl.pallas_call(
        _matmul_kernel,
        out_shape=jax.ShapeDtypeStruct((M, N), out_dtype),
        grid=(grid_m,),
        in_specs=[
            pl.BlockSpec((tm, K), lambda i: (i, 0)),
            pl.BlockSpec((N, K), lambda i: (0, 0)),
        ],
        out_specs=pl.BlockSpec((tm, N), lambda i: (i, 0)),
        scratch_shapes=[pltpu.VMEM((K, N), jnp.bfloat16)],
        compiler_params=pltpu.CompilerParams(
            dimension_semantics=("arbitrary",),
            vmem_limit_bytes=int(footprint + 8 * _MIB),
        ),
        cost_estimate=pl.CostEstimate(
            flops=2 * M * N * K,
            transcendentals=0,
            bytes_accessed=M * K * 4 + K * N * 4 + M * N * 4,
        ),
    )(x, weight)


# speedup vs baseline: 1.0156x; 1.0156x over previous
import jax
import jax.numpy as jnp
from jax.experimental import pallas as pl
from jax.experimental.pallas import tpu as pltpu

_MIB = 1024 * 1024


def _matmul_kernel(x_ref, w_ref, o_ref, w_bf_ref):
    @pl.when(pl.program_id(0) == 0)
    def _():
        w_bf_ref[...] = w_ref[...].T.astype(jnp.bfloat16)

    o_ref[...] = jnp.dot(
        x_ref[...].astype(jnp.bfloat16),
        w_bf_ref[...],
        preferred_element_type=jnp.float32,
    )


def kernel(x, weight):
    M, K = x.shape
    N = weight.shape[0]
    out_dtype = x.dtype

    tm = 512
    grid_m = M // tm

    footprint = K * N * 4 + K * N * 2 + 2 * tm * (K + N) * 4

    return pl.pallas_call(
        _matmul_kernel,
        out_shape=jax.ShapeDtypeStruct((M, N), out_dtype),
        grid=(grid_m,),
        in_specs=[
            pl.BlockSpec((tm, K), lambda i: (i, 0)),
            pl.BlockSpec((N, K), lambda i: (0, 0)),
        ],
        out_specs=pl.BlockSpec((tm, N), lambda i: (i, 0)),
        scratch_shapes=[pltpu.VMEM((K, N), jnp.bfloat16)],
        compiler_params=pltpu.CompilerParams(
            dimension_semantics=("arbitrary",),
            vmem_limit_bytes=int(footprint + 14 * _MIB),
        ),
        cost_estimate=pl.CostEstimate(
            flops=2 * M * N * K,
            transcendentals=0,
            bytes_accessed=M * K * 4 + K * N * 4 + M * N * 4,
        ),
    )(x, weight)


# speedup vs baseline: 1.0157x; 1.0001x over previous
import jax
import jax.numpy as jnp
from jax.experimental import pallas as pl
from jax.experimental.pallas import tpu as pltpu

_MIB = 1024 * 1024
_B = 256


def _matmul_kernel(x_ref, w_ref, o_ref, w_bf_ref):
    @pl.when(pl.program_id(0) == 0)
    def _():
        w_bf_ref[...] = w_ref[...].T.astype(jnp.bfloat16)

    x_bf = x_ref[...].astype(jnp.bfloat16)
    tm = x_ref.shape[0]
    k_groups = x_ref.shape[1] // _B
    n_groups = o_ref.shape[1] // _B

    for n in range(n_groups):
        mxu = n % 2
        acc = ((n // 2) % 2) * (tm // 4)
        for k in range(k_groups):
            sr = k % 2
            pltpu.matmul_push_rhs(
                w_bf_ref[k * _B:(k + 1) * _B, n * _B:(n + 1) * _B],
                staging_register=sr,
                mxu_index=mxu,
            )
            pltpu.matmul_acc_lhs(
                acc,
                x_bf[:, k * _B:(k + 1) * _B],
                mxu_index=mxu,
                load_staged_rhs=sr,
            )
        o_ref[:, n * _B:(n + 1) * _B] = pltpu.matmul_pop(
            acc, (tm, _B), jnp.float32, mxu
        )


def kernel(x, weight):
    M, K = x.shape
    N = weight.shape[0]
    out_dtype = x.dtype

    tm = 512
    grid_m = M // tm

    footprint = K * N * 4 + K * N * 2 + 2 * tm * (K + N) * 4

    return pl.pallas_call(
        _matmul_kernel,
        out_shape=jax.ShapeDtypeStruct((M, N), out_dtype),
        grid=(grid_m,),
        in_specs=[
            pl.BlockSpec((tm, K), lambda i: (i, 0)),
            pl.BlockSpec((N, K), lambda i: (0, 0)),
        ],
        out_specs=pl.BlockSpec((tm, N), lambda i: (i, 0)),
        scratch_shapes=[pltpu.VMEM((K, N), jnp.bfloat16)],
        compiler_params=pltpu.CompilerParams(
            dimension_semantics=("arbitrary",),
            vmem_limit_bytes=int(footprint + 14 * _MIB),
        ),
        cost_estimate=pl.CostEstimate(
            flops=2 * M * N * K,
            transcendentals=0,
            bytes_accessed=M * K * 4 + K * N * 4 + M * N * 4,
        ),
    )(x, weight)


# speedup vs baseline: 1.0166x; 1.0009x over previous
import jax
import jax.numpy as jnp
from jax.experimental import pallas as pl
from jax.experimental.pallas import tpu as pltpu

_MIB = 1024 * 1024


def _matmul_kernel(x_ref, w_ref, o_ref, w_bf_ref):
    @pl.when(pl.program_id(0) == 0)
    def _():
        w_bf_ref[...] = w_ref[...].T.astype(jnp.bfloat16)

    o_ref[...] = jnp.dot(
        x_ref[...].astype(jnp.bfloat16),
        w_bf_ref[...],
        preferred_element_type=jnp.float32,
    )


def kernel(x, weight):
    M, K = x.shape
    N = weight.shape[0]
    out_dtype = x.dtype

    tm = 512
    grid_m = M // tm

    footprint = K * N * 4 + K * N * 2 + 2 * tm * (K + N) * 4

    return pl.pallas_call(
        _matmul_kernel,
        out_shape=jax.ShapeDtypeStruct((M, N), out_dtype),
        grid=(grid_m,),
        in_specs=[
            pl.BlockSpec((tm, K), lambda i: (i, 0)),
            pl.BlockSpec((N, K), lambda i: (0, 0)),
        ],
        out_specs=pl.BlockSpec((tm, N), lambda i: (i, 0)),
        scratch_shapes=[pltpu.VMEM((K, N), jnp.bfloat16)],
        compiler_params=pltpu.CompilerParams(
            dimension_semantics=("arbitrary",),
            vmem_limit_bytes=int(footprint + 14 * _MIB),
        ),
        cost_estimate=pl.CostEstimate(
            flops=2 * M * N * K,
            transcendentals=0,
            bytes_accessed=M * K * 4 + K * N * 4 + M * N * 4,
        ),
    )(x, weight)
